# baseline (speedup 1.0000x reference)
import functools

import jax
import jax.numpy as jnp
from jax import lax
from jax.experimental import pallas as pl
from jax.experimental.pallas import tpu as pltpu
from jax.experimental.pallas import tpu_sc as plsc

N = 10000
E = 320000
D = 128
NC, NS, L = 2, 16, 16
NW = NC * NS
EPT = 10240
CH = 128
NCH = EPT // CH
GRP = 8
NG = NCH // GRP
E_PAD = NW * EPT
NCHT = E_PAD // CH
NGRP_TOT = NCHT // GRP
CORE_FAST = 1
CH_FAST = 160
CH_SLOW = 0
N_ACC = 10112
STRIPE = N_ACC // NS
BLK = 632


def _sc_agg_body(h, src, dst, zeros_a, part_out,
                 src_v, dst_v, rows_v, acc_s, gsem):
    c = lax.axis_index("c")
    s = lax.axis_index("s")
    base = s * STRIPE

    cnt = jnp.where(c == CORE_FAST, CH_FAST, CH_SLOW)
    grp0 = (jnp.where(c == CORE_FAST, 0, NS * CH_FAST) + s * cnt) // GRP
    grp0 = jnp.minimum(grp0, NGRP_TOT - 1)
    ngrp = cnt // GRP

    pltpu.sync_copy(zeros_a, acc_s.at[pl.ds(base, STRIPE)])

    pltpu.sync_copy(src.at[grp0], src_v.at[0])
    pltpu.sync_copy(dst.at[grp0], dst_v.at[0])

    plsc.subcore_barrier()

    @pl.when(cnt > 0)
    def _():
        pltpu.async_copy(h.at[src_v.at[0, 0]], rows_v.at[0], gsem)

    def _group(g, _):
        gm = lax.rem(g, 2)
        gp = lax.rem(g + 1, 2)

        @pl.when(g + 1 < ngrp)
        def _():
            pltpu.sync_copy(src.at[grp0 + g + 1], src_v.at[gp])
            pltpu.sync_copy(dst.at[grp0 + g + 1], dst_v.at[gp])

        for k in range(GRP):
            j = g * GRP + k
            pltpu.make_async_copy(
                h.at[src_v.at[gm, k]], rows_v.at[k % 2], gsem).wait()

            @pl.when(j + 1 < cnt)
            def _():
                if k + 1 < GRP:
                    nidx = src_v.at[gm, k + 1]
                else:
                    nidx = src_v.at[gp, 0]
                pltpu.async_copy(h.at[nidx], rows_v.at[(k + 1) % 2], gsem)

            pltpu.sync_copy(rows_v.at[k % 2], acc_s.at[dst_v.at[gm, k]],
                            add=True)
        return 0
    lax.fori_loop(0, ngrp, _group, 0)

    plsc.subcore_barrier()

    pltpu.sync_copy(acc_s.at[pl.ds(base, STRIPE)],
                    part_out.at[c, pl.ds(base, STRIPE)])


def _sc_deg_body(dst, zeros_b, ones_in, deg_out,
                 dst_v, ones_v, dega_s, _sem):
    c = lax.axis_index("c")
    s = lax.axis_index("s")
    wid = c * NS + s
    base = s * STRIPE

    pltpu.sync_copy(zeros_b, dega_s.at[pl.ds(base, STRIPE)])
    pltpu.sync_copy(ones_in, ones_v)
    pltpu.sync_copy(dst.at[wid], dst_v)

    plsc.subcore_barrier()

    def _chunk(j, _):
        pltpu.sync_copy(ones_v, dega_s.at[dst_v.at[j]], add=True)
        return 0
    lax.fori_loop(0, NCH, _chunk, 0)

    plsc.subcore_barrier()

    pltpu.sync_copy(dega_s.at[pl.ds(base, STRIPE)],
                    deg_out.at[c, pl.ds(base, STRIPE)])


def _make_sc_agg():
    mesh = plsc.VectorSubcoreMesh(core_axis_name="c", subcore_axis_name="s")
    return pl.kernel(
        _sc_agg_body,
        out_type=jax.ShapeDtypeStruct((NC, N_ACC, D), jnp.float32),
        mesh=mesh,
        scratch_types=(
            pltpu.VMEM((2, GRP, CH), jnp.int32),
            pltpu.VMEM((2, GRP, CH), jnp.int32),
            pltpu.VMEM((2, CH, D), jnp.float32),
            pltpu.VMEM_SHARED((N_ACC, D), jnp.float32),
            pltpu.SemaphoreType.DMA,
        ),
    )


def _make_sc_deg():
    mesh = plsc.VectorSubcoreMesh(core_axis_name="c", subcore_axis_name="s")
    return pl.kernel(
        _sc_deg_body,
        out_type=jax.ShapeDtypeStruct((NC, N_ACC, D), jnp.float32),
        mesh=mesh,
        scratch_types=(
            pltpu.VMEM((NCH, CH), jnp.int32),
            pltpu.VMEM((CH, D), jnp.float32),
            pltpu.VMEM_SHARED((N_ACC, D), jnp.float32),
            pltpu.SemaphoreType.DMA,
        ),
    )


def _dense_body(relu, h_ref, p_ref, d0_ref, d1_ref, ws_ref, wn_ref, b_ref, o_ref):
    deg = d0_ref[...] + d1_ref[...]
    recip = 1.0 / jnp.maximum(deg, 1.0)
    ssum = p_ref[0] + p_ref[1]
    hn = ssum * recip
    z = (jnp.dot(h_ref[...], ws_ref[...], preferred_element_type=jnp.float32)
         + jnp.dot(hn, wn_ref[...], preferred_element_type=jnp.float32)
         + b_ref[...])
    o_ref[...] = jnp.maximum(z, 0.0) if relu else z


def _dense(h, part, d0, d1, Ws, Wn, b, relu):
    return pl.pallas_call(
        functools.partial(_dense_body, relu),
        grid=(NS,),
        in_specs=[
            pl.BlockSpec((BLK, D), lambda i: (i, 0)),
            pl.BlockSpec((NC, BLK, D), lambda i: (0, i, 0)),
            pl.BlockSpec((BLK, 1), lambda i: (i, 0)),
            pl.BlockSpec((BLK, 1), lambda i: (i, 0)),
            pl.BlockSpec((D, D), lambda i: (0, 0)),
            pl.BlockSpec((D, D), lambda i: (0, 0)),
            pl.BlockSpec((1, D), lambda i: (0, 0)),
        ],
        out_specs=pl.BlockSpec((BLK, D), lambda i: (i, 0)),
        out_shape=jax.ShapeDtypeStruct((N, D), jnp.float32),
    )(h, part, d0, d1, Ws, Wn, b)


_sc_agg = _make_sc_agg()
_sc_deg = _make_sc_deg()


def kernel(x, edge_index, W_self1, W_neigh1, b1, W_self2, W_neigh2, b2):
    src = edge_index[0]
    dst = edge_index[1]
    src_p = jnp.concatenate(
        [src, jnp.zeros((E_PAD - E,), jnp.int32)]).reshape(NW, NCH, CH)
    dst_p = jnp.concatenate(
        [dst, jnp.full((E_PAD - E,), N_ACC - 1, jnp.int32)]).reshape(NW, NCH, CH)

    zeros_a = jnp.zeros((STRIPE, D), jnp.float32)
    ones_c = jnp.ones((CH, D), jnp.float32)

    degp = _sc_deg(dst_p, zeros_a, ones_c)
    d0 = degp[0, :, 0:1]
    d1 = degp[1, :, 0:1]
    b1r = b1.reshape(1, D)
    b2r = b2.reshape(1, D)

    src_g = src_p.reshape(NGRP_TOT, GRP, CH)
    dst_g = dst_p.reshape(NGRP_TOT, GRP, CH)

    part1 = _sc_agg(x, src_g, dst_g, zeros_a)
    h1 = _dense(x, part1, d0, d1, W_self1, W_neigh1, b1r, relu=True)
    part2 = _sc_agg(h1, src_g, dst_g, zeros_a)
    out = _dense(h1, part2, d0, d1, W_self2, W_neigh2, b2r, relu=False)
    return out


# speedup vs baseline: 1.3495x; 1.3495x over previous
import functools

import jax
import jax.numpy as jnp
from jax import lax
from jax.experimental import pallas as pl
from jax.experimental.pallas import tpu as pltpu
from jax.experimental.pallas import tpu_sc as plsc

N = 10000
E = 320000
D = 128
NC, NS, L = 2, 16, 16
NW = NC * NS
EPT = 10240
CH = 128
NCH = EPT // CH
GRP = 8
NG = NCH // GRP
E_PAD = NW * EPT
NCHT = E_PAD // CH
NGRP_TOT = NCHT // GRP
CORE_FAST = 1
CH_FAST = 152
CH_SLOW = 8
N_ACC = 10112
STRIPE = N_ACC // NS
BLK = 632


def _sc_agg_body(h, src, dst, zeros_a, part_out,
                 src_v, dst_v, rows_v, acc_s, gsem):
    c = lax.axis_index("c")
    s = lax.axis_index("s")
    base = s * STRIPE

    cnt = jnp.where(c == CORE_FAST, CH_FAST, CH_SLOW)
    grp0 = (jnp.where(c == CORE_FAST, 0, NS * CH_FAST) + s * cnt) // GRP
    grp0 = jnp.minimum(grp0, NGRP_TOT - 1)
    ngrp = cnt // GRP

    pltpu.sync_copy(zeros_a, acc_s.at[pl.ds(base, STRIPE)])

    pltpu.sync_copy(src.at[grp0], src_v.at[0])
    pltpu.sync_copy(dst.at[grp0], dst_v.at[0])

    plsc.subcore_barrier()

    @pl.when(cnt > 0)
    def _():
        pltpu.async_copy(h.at[src_v.at[0, 0]], rows_v.at[0], gsem)

    def _group(g, _):
        gm = lax.rem(g, 2)
        gp = lax.rem(g + 1, 2)

        @pl.when(g + 1 < ngrp)
        def _():
            pltpu.sync_copy(src.at[grp0 + g + 1], src_v.at[gp])
            pltpu.sync_copy(dst.at[grp0 + g + 1], dst_v.at[gp])

        for k in range(GRP):
            j = g * GRP + k
            pltpu.make_async_copy(
                h.at[src_v.at[gm, k]], rows_v.at[k % 2], gsem).wait()

            @pl.when(j + 1 < cnt)
            def _():
                if k + 1 < GRP:
                    nidx = src_v.at[gm, k + 1]
                else:
                    nidx = src_v.at[gp, 0]
                pltpu.async_copy(h.at[nidx], rows_v.at[(k + 1) % 2], gsem)

            pltpu.sync_copy(rows_v.at[k % 2], acc_s.at[dst_v.at[gm, k]],
                            add=True)
        return 0
    lax.fori_loop(0, ngrp, _group, 0)

    plsc.subcore_barrier()

    pltpu.sync_copy(acc_s.at[pl.ds(base, STRIPE)],
                    part_out.at[c, pl.ds(base, STRIPE)])


def _sc_deg_body(dst, zeros_b, ones_in, deg_out,
                 dst_v, ones_v, dega_s, _sem):
    c = lax.axis_index("c")
    s = lax.axis_index("s")
    wid = c * NS + s
    base = s * STRIPE

    pltpu.sync_copy(zeros_b, dega_s.at[pl.ds(base, STRIPE)])
    pltpu.sync_copy(ones_in, ones_v)
    pltpu.sync_copy(dst.at[wid], dst_v)

    plsc.subcore_barrier()

    def _chunk(j, _):
        pltpu.sync_copy(ones_v, dega_s.at[dst_v.at[j]], add=True)
        return 0
    lax.fori_loop(0, NCH, _chunk, 0)

    plsc.subcore_barrier()

    pltpu.sync_copy(dega_s.at[pl.ds(base, STRIPE)],
                    deg_out.at[c, pl.ds(base, STRIPE)])


def _make_sc_agg():
    mesh = plsc.VectorSubcoreMesh(core_axis_name="c", subcore_axis_name="s")
    return pl.kernel(
        _sc_agg_body,
        out_type=jax.ShapeDtypeStruct((NC, N_ACC, D), jnp.float32),
        mesh=mesh,
        scratch_types=(
            pltpu.VMEM((2, GRP, CH), jnp.int32),
            pltpu.VMEM((2, GRP, CH), jnp.int32),
            pltpu.VMEM((2, CH, D), jnp.float32),
            pltpu.VMEM_SHARED((N_ACC, D), jnp.float32),
            pltpu.SemaphoreType.DMA,
        ),
    )


def _make_sc_deg():
    mesh = plsc.VectorSubcoreMesh(core_axis_name="c", subcore_axis_name="s")
    return pl.kernel(
        _sc_deg_body,
        out_type=jax.ShapeDtypeStruct((NC, N_ACC, D), jnp.float32),
        mesh=mesh,
        scratch_types=(
            pltpu.VMEM((NCH, CH), jnp.int32),
            pltpu.VMEM((CH, D), jnp.float32),
            pltpu.VMEM_SHARED((N_ACC, D), jnp.float32),
            pltpu.SemaphoreType.DMA,
        ),
    )


def _dense_body(relu, h_ref, p_ref, d0_ref, d1_ref, ws_ref, wn_ref, b_ref, o_ref):
    deg = d0_ref[...] + d1_ref[...]
    recip = 1.0 / jnp.maximum(deg, 1.0)
    ssum = p_ref[0] + p_ref[1]
    hn = ssum * recip
    z = (jnp.dot(h_ref[...], ws_ref[...], preferred_element_type=jnp.float32)
         + jnp.dot(hn, wn_ref[...], preferred_element_type=jnp.float32)
         + b_ref[...])
    o_ref[...] = jnp.maximum(z, 0.0) if relu else z


def _dense(h, part, d0, d1, Ws, Wn, b, relu):
    return pl.pallas_call(
        functools.partial(_dense_body, relu),
        grid=(NS,),
        in_specs=[
            pl.BlockSpec((BLK, D), lambda i: (i, 0)),
            pl.BlockSpec((NC, BLK, D), lambda i: (0, i, 0)),
            pl.BlockSpec((BLK, 1), lambda i: (i, 0)),
            pl.BlockSpec((BLK, 1), lambda i: (i, 0)),
            pl.BlockSpec((D, D), lambda i: (0, 0)),
            pl.BlockSpec((D, D), lambda i: (0, 0)),
            pl.BlockSpec((1, D), lambda i: (0, 0)),
        ],
        out_specs=pl.BlockSpec((BLK, D), lambda i: (i, 0)),
        out_shape=jax.ShapeDtypeStruct((N, D), jnp.float32),
    )(h, part, d0, d1, Ws, Wn, b)


_sc_agg = _make_sc_agg()
_sc_deg = _make_sc_deg()


def kernel(x, edge_index, W_self1, W_neigh1, b1, W_self2, W_neigh2, b2):
    src = edge_index[0]
    dst = edge_index[1]
    src_p = jnp.concatenate(
        [src, jnp.zeros((E_PAD - E,), jnp.int32)]).reshape(NW, NCH, CH)
    dst_p = jnp.concatenate(
        [dst, jnp.full((E_PAD - E,), N_ACC - 1, jnp.int32)]).reshape(NW, NCH, CH)

    zeros_a = jnp.zeros((STRIPE, D), jnp.float32)
    ones_c = jnp.ones((CH, D), jnp.float32)

    degp = _sc_deg(dst_p, zeros_a, ones_c)
    d0 = degp[0, :, 0:1]
    d1 = degp[1, :, 0:1]
    b1r = b1.reshape(1, D)
    b2r = b2.reshape(1, D)

    src_g = src_p.reshape(NGRP_TOT, GRP, CH)
    dst_g = dst_p.reshape(NGRP_TOT, GRP, CH)

    part1 = _sc_agg(x, src_g, dst_g, zeros_a)
    h1 = _dense(x, part1, d0, d1, W_self1, W_neigh1, b1r, relu=True)
    part2 = _sc_agg(h1, src_g, dst_g, zeros_a)
    out = _dense(h1, part2, d0, d1, W_self2, W_neigh2, b2r, relu=False)
    return out


# speedup vs baseline: 1.3616x; 1.0089x over previous
import functools

import jax
import jax.numpy as jnp
from jax import lax
from jax.experimental import pallas as pl
from jax.experimental.pallas import tpu as pltpu
from jax.experimental.pallas import tpu_sc as plsc

N = 10000
E = 320000
D = 128
NC, NS, L = 2, 16, 16
NW = NC * NS
EPT = 10240
CH = 128
NCH = EPT // CH
GRP = 8
NG = NCH // GRP
E_PAD = NW * EPT
NCHT = E_PAD // CH
NGRP_TOT = NCHT // GRP
CORE_FAST = 1
CH_FAST = 144
CH_SLOW = 16
N_ACC = 10112
STRIPE = N_ACC // NS
BLK = 632


def _sc_agg_body(h, src, dst, zeros_a, part_out,
                 src_v, dst_v, rows_v, acc_s, gsem):
    c = lax.axis_index("c")
    s = lax.axis_index("s")
    base = s * STRIPE

    cnt = jnp.where(c == CORE_FAST, CH_FAST, CH_SLOW)
    grp0 = (jnp.where(c == CORE_FAST, 0, NS * CH_FAST) + s * cnt) // GRP
    grp0 = jnp.minimum(grp0, NGRP_TOT - 1)
    ngrp = cnt // GRP

    pltpu.sync_copy(zeros_a, acc_s.at[pl.ds(base, STRIPE)])

    pltpu.sync_copy(src.at[grp0], src_v.at[0])
    pltpu.sync_copy(dst.at[grp0], dst_v.at[0])

    plsc.subcore_barrier()

    @pl.when(cnt > 0)
    def _():
        pltpu.async_copy(h.at[src_v.at[0, 0]], rows_v.at[0], gsem)

    def _group(g, _):
        gm = lax.rem(g, 2)
        gp = lax.rem(g + 1, 2)

        @pl.when(g + 1 < ngrp)
        def _():
            pltpu.sync_copy(src.at[grp0 + g + 1], src_v.at[gp])
            pltpu.sync_copy(dst.at[grp0 + g + 1], dst_v.at[gp])

        for k in range(GRP):
            j = g * GRP + k
            pltpu.make_async_copy(
                h.at[src_v.at[gm, k]], rows_v.at[k % 2], gsem).wait()

            @pl.when(j + 1 < cnt)
            def _():
                if k + 1 < GRP:
                    nidx = src_v.at[gm, k + 1]
                else:
                    nidx = src_v.at[gp, 0]
                pltpu.async_copy(h.at[nidx], rows_v.at[(k + 1) % 2], gsem)

            pltpu.sync_copy(rows_v.at[k % 2], acc_s.at[dst_v.at[gm, k]],
                            add=True)
        return 0
    lax.fori_loop(0, ngrp, _group, 0)

    plsc.subcore_barrier()

    pltpu.sync_copy(acc_s.at[pl.ds(base, STRIPE)],
                    part_out.at[c, pl.ds(base, STRIPE)])


def _sc_deg_body(dst, zeros_b, ones_in, deg_out,
                 dst_v, ones_v, dega_s, _sem):
    c = lax.axis_index("c")
    s = lax.axis_index("s")
    wid = c * NS + s
    base = s * STRIPE

    pltpu.sync_copy(zeros_b, dega_s.at[pl.ds(base, STRIPE)])
    pltpu.sync_copy(ones_in, ones_v)
    pltpu.sync_copy(dst.at[wid], dst_v)

    plsc.subcore_barrier()

    def _chunk(j, _):
        pltpu.sync_copy(ones_v, dega_s.at[dst_v.at[j]], add=True)
        return 0
    lax.fori_loop(0, NCH, _chunk, 0)

    plsc.subcore_barrier()

    pltpu.sync_copy(dega_s.at[pl.ds(base, STRIPE)],
                    deg_out.at[c, pl.ds(base, STRIPE)])


def _make_sc_agg():
    mesh = plsc.VectorSubcoreMesh(core_axis_name="c", subcore_axis_name="s")
    return pl.kernel(
        _sc_agg_body,
        out_type=jax.ShapeDtypeStruct((NC, N_ACC, D), jnp.float32),
        mesh=mesh,
        scratch_types=(
            pltpu.VMEM((2, GRP, CH), jnp.int32),
            pltpu.VMEM((2, GRP, CH), jnp.int32),
            pltpu.VMEM((2, CH, D), jnp.float32),
            pltpu.VMEM_SHARED((N_ACC, D), jnp.float32),
            pltpu.SemaphoreType.DMA,
        ),
    )


def _make_sc_deg():
    mesh = plsc.VectorSubcoreMesh(core_axis_name="c", subcore_axis_name="s")
    return pl.kernel(
        _sc_deg_body,
        out_type=jax.ShapeDtypeStruct((NC, N_ACC, D), jnp.float32),
        mesh=mesh,
        scratch_types=(
            pltpu.VMEM((NCH, CH), jnp.int32),
            pltpu.VMEM((CH, D), jnp.float32),
            pltpu.VMEM_SHARED((N_ACC, D), jnp.float32),
            pltpu.SemaphoreType.DMA,
        ),
    )


def _dense_body(relu, h_ref, p_ref, d0_ref, d1_ref, ws_ref, wn_ref, b_ref, o_ref):
    deg = d0_ref[...] + d1_ref[...]
    recip = 1.0 / jnp.maximum(deg, 1.0)
    ssum = p_ref[0] + p_ref[1]
    hn = ssum * recip
    z = (jnp.dot(h_ref[...], ws_ref[...], preferred_element_type=jnp.float32)
         + jnp.dot(hn, wn_ref[...], preferred_element_type=jnp.float32)
         + b_ref[...])
    o_ref[...] = jnp.maximum(z, 0.0) if relu else z


def _dense(h, part, d0, d1, Ws, Wn, b, relu):
    return pl.pallas_call(
        functools.partial(_dense_body, relu),
        grid=(NS,),
        in_specs=[
            pl.BlockSpec((BLK, D), lambda i: (i, 0)),
            pl.BlockSpec((NC, BLK, D), lambda i: (0, i, 0)),
            pl.BlockSpec((BLK, 1), lambda i: (i, 0)),
            pl.BlockSpec((BLK, 1), lambda i: (i, 0)),
            pl.BlockSpec((D, D), lambda i: (0, 0)),
            pl.BlockSpec((D, D), lambda i: (0, 0)),
            pl.BlockSpec((1, D), lambda i: (0, 0)),
        ],
        out_specs=pl.BlockSpec((BLK, D), lambda i: (i, 0)),
        out_shape=jax.ShapeDtypeStruct((N, D), jnp.float32),
    )(h, part, d0, d1, Ws, Wn, b)


_sc_agg = _make_sc_agg()
_sc_deg = _make_sc_deg()


def kernel(x, edge_index, W_self1, W_neigh1, b1, W_self2, W_neigh2, b2):
    src = edge_index[0]
    dst = edge_index[1]
    src_p = jnp.concatenate(
        [src, jnp.zeros((E_PAD - E,), jnp.int32)]).reshape(NW, NCH, CH)
    dst_p = jnp.concatenate(
        [dst, jnp.full((E_PAD - E,), N_ACC - 1, jnp.int32)]).reshape(NW, NCH, CH)

    zeros_a = jnp.zeros((STRIPE, D), jnp.float32)
    ones_c = jnp.ones((CH, D), jnp.float32)

    degp = _sc_deg(dst_p, zeros_a, ones_c)
    d0 = degp[0, :, 0:1]
    d1 = degp[1, :, 0:1]
    b1r = b1.reshape(1, D)
    b2r = b2.reshape(1, D)

    src_g = src_p.reshape(NGRP_TOT, GRP, CH)
    dst_g = dst_p.reshape(NGRP_TOT, GRP, CH)

    part1 = _sc_agg(x, src_g, dst_g, zeros_a)
    h1 = _dense(x, part1, d0, d1, W_self1, W_neigh1, b1r, relu=True)
    part2 = _sc_agg(h1, src_g, dst_g, zeros_a)
    out = _dense(h1, part2, d0, d1, W_self2, W_neigh2, b2r, relu=False)
    return out


# speedup vs baseline: 1.3636x; 1.0015x over previous
import functools

import jax
import jax.numpy as jnp
from jax import lax
from jax.experimental import pallas as pl
from jax.experimental.pallas import tpu as pltpu
from jax.experimental.pallas import tpu_sc as plsc

N = 10000
E = 320000
D = 128
NC, NS, L = 2, 16, 16
NW = NC * NS
EPT = 10240
CH = 128
NCH = EPT // CH
GRP = 8
NG = NCH // GRP
E_PAD = NW * EPT
NCHT = E_PAD // CH
NGRP_TOT = NCHT // GRP
CORE_FAST = 1
CH_FAST = 144
CH_SLOW = 16
N_ACC = 10112
STRIPE = N_ACC // NS
BLK = 632


def _sc_agg_body(h, src, dst, zeros_a, part_out,
                 src_v, dst_v, rows_v, acc_s, gsem):
    c = lax.axis_index("c")
    s = lax.axis_index("s")
    base = s * STRIPE

    cnt = jnp.where(c == CORE_FAST, CH_FAST, CH_SLOW)
    grp0 = (jnp.where(c == CORE_FAST, 0, NS * CH_FAST) + s * cnt) // GRP
    grp0 = jnp.minimum(grp0, NGRP_TOT - 1)
    ngrp = cnt // GRP

    pltpu.sync_copy(zeros_a, acc_s.at[pl.ds(base, STRIPE)])

    pltpu.sync_copy(src.at[grp0], src_v.at[0])
    pltpu.sync_copy(dst.at[grp0], dst_v.at[0])

    plsc.subcore_barrier()

    @pl.when(cnt > 0)
    def _():
        pltpu.async_copy(h.at[src_v.at[0, 0]], rows_v.at[0], gsem)

    def _group(g, _):
        gm = lax.rem(g, 2)
        gp = lax.rem(g + 1, 2)

        @pl.when(g + 1 < ngrp)
        def _():
            pltpu.sync_copy(src.at[grp0 + g + 1], src_v.at[gp])
            pltpu.sync_copy(dst.at[grp0 + g + 1], dst_v.at[gp])

        for k in range(GRP):
            j = g * GRP + k

            @pl.when(j + 1 < cnt)
            def _():
                if k + 1 < GRP:
                    nidx = src_v.at[gm, k + 1]
                else:
                    nidx = src_v.at[gp, 0]
                pltpu.async_copy(h.at[nidx], rows_v.at[(k + 1) % 2], gsem)

            pltpu.make_async_copy(
                h.at[src_v.at[gm, k]], rows_v.at[k % 2], gsem).wait()

            pltpu.sync_copy(rows_v.at[k % 2], acc_s.at[dst_v.at[gm, k]],
                            add=True)
        return 0
    lax.fori_loop(0, ngrp, _group, 0)

    plsc.subcore_barrier()

    pltpu.sync_copy(acc_s.at[pl.ds(base, STRIPE)],
                    part_out.at[c, pl.ds(base, STRIPE)])


def _sc_deg_body(dst, zeros_b, ones_in, deg_out,
                 dst_v, ones_v, dega_s, _sem):
    c = lax.axis_index("c")
    s = lax.axis_index("s")
    wid = c * NS + s
    base = s * STRIPE

    pltpu.sync_copy(zeros_b, dega_s.at[pl.ds(base, STRIPE)])
    pltpu.sync_copy(ones_in, ones_v)
    pltpu.sync_copy(dst.at[wid], dst_v)

    plsc.subcore_barrier()

    def _chunk(j, _):
        pltpu.sync_copy(ones_v, dega_s.at[dst_v.at[j]], add=True)
        return 0
    lax.fori_loop(0, NCH, _chunk, 0)

    plsc.subcore_barrier()

    pltpu.sync_copy(dega_s.at[pl.ds(base, STRIPE)],
                    deg_out.at[c, pl.ds(base, STRIPE)])


def _make_sc_agg():
    mesh = plsc.VectorSubcoreMesh(core_axis_name="c", subcore_axis_name="s")
    return pl.kernel(
        _sc_agg_body,
        out_type=jax.ShapeDtypeStruct((NC, N_ACC, D), jnp.float32),
        mesh=mesh,
        scratch_types=(
            pltpu.VMEM((2, GRP, CH), jnp.int32),
            pltpu.VMEM((2, GRP, CH), jnp.int32),
            pltpu.VMEM((2, CH, D), jnp.float32),
            pltpu.VMEM_SHARED((N_ACC, D), jnp.float32),
            pltpu.SemaphoreType.DMA,
        ),
    )


def _make_sc_deg():
    mesh = plsc.VectorSubcoreMesh(core_axis_name="c", subcore_axis_name="s")
    return pl.kernel(
        _sc_deg_body,
        out_type=jax.ShapeDtypeStruct((NC, N_ACC, D), jnp.float32),
        mesh=mesh,
        scratch_types=(
            pltpu.VMEM((NCH, CH), jnp.int32),
            pltpu.VMEM((CH, D), jnp.float32),
            pltpu.VMEM_SHARED((N_ACC, D), jnp.float32),
            pltpu.SemaphoreType.DMA,
        ),
    )


def _dense_body(relu, h_ref, p_ref, d0_ref, d1_ref, ws_ref, wn_ref, b_ref, o_ref):
    deg = d0_ref[...] + d1_ref[...]
    recip = 1.0 / jnp.maximum(deg, 1.0)
    ssum = p_ref[0] + p_ref[1]
    hn = ssum * recip
    z = (jnp.dot(h_ref[...], ws_ref[...], preferred_element_type=jnp.float32)
         + jnp.dot(hn, wn_ref[...], preferred_element_type=jnp.float32)
         + b_ref[...])
    o_ref[...] = jnp.maximum(z, 0.0) if relu else z


def _dense(h, part, d0, d1, Ws, Wn, b, relu):
    return pl.pallas_call(
        functools.partial(_dense_body, relu),
        grid=(NS,),
        in_specs=[
            pl.BlockSpec((BLK, D), lambda i: (i, 0)),
            pl.BlockSpec((NC, BLK, D), lambda i: (0, i, 0)),
            pl.BlockSpec((BLK, 1), lambda i: (i, 0)),
            pl.BlockSpec((BLK, 1), lambda i: (i, 0)),
            pl.BlockSpec((D, D), lambda i: (0, 0)),
            pl.BlockSpec((D, D), lambda i: (0, 0)),
            pl.BlockSpec((1, D), lambda i: (0, 0)),
        ],
        out_specs=pl.BlockSpec((BLK, D), lambda i: (i, 0)),
        out_shape=jax.ShapeDtypeStruct((N, D), jnp.float32),
    )(h, part, d0, d1, Ws, Wn, b)


_sc_agg = _make_sc_agg()
_sc_deg = _make_sc_deg()


def kernel(x, edge_index, W_self1, W_neigh1, b1, W_self2, W_neigh2, b2):
    src = edge_index[0]
    dst = edge_index[1]
    src_p = jnp.concatenate(
        [src, jnp.zeros((E_PAD - E,), jnp.int32)]).reshape(NW, NCH, CH)
    dst_p = jnp.concatenate(
        [dst, jnp.full((E_PAD - E,), N_ACC - 1, jnp.int32)]).reshape(NW, NCH, CH)

    zeros_a = jnp.zeros((STRIPE, D), jnp.float32)
    ones_c = jnp.ones((CH, D), jnp.float32)

    degp = _sc_deg(dst_p, zeros_a, ones_c)
    d0 = degp[0, :, 0:1]
    d1 = degp[1, :, 0:1]
    b1r = b1.reshape(1, D)
    b2r = b2.reshape(1, D)

    src_g = src_p.reshape(NGRP_TOT, GRP, CH)
    dst_g = dst_p.reshape(NGRP_TOT, GRP, CH)

    part1 = _sc_agg(x, src_g, dst_g, zeros_a)
    h1 = _dense(x, part1, d0, d1, W_self1, W_neigh1, b1r, relu=True)
    part2 = _sc_agg(h1, src_g, dst_g, zeros_a)
    out = _dense(h1, part2, d0, d1, W_self2, W_neigh2, b2r, relu=False)
    return out
